# baseline (speedup 1.0000x reference)
import functools

import jax
import jax.numpy as jnp
from jax import lax
from jax.experimental import pallas as pl
from jax.experimental.pallas import tpu as pltpu
from jax.experimental.pallas import tpu_sc as plsc

B = 16384
D = 16
NC = 2
NS = 16
NW = NC * NS
ROWS = B // NW
CH = 128
NCH = ROWS // CH


def _irt_body(theta_hbm, alpha_hbm, beta_hbm, sidx_hbm, qidx_hbm, out_hbm,
              sidx_v, qidx_v, theta_v, alpha_v, beta_v, out_v, sem):
    wid = lax.axis_index("s") * NC + lax.axis_index("c")
    base = wid * ROWS

    pltpu.sync_copy(sidx_hbm.at[wid], sidx_v)
    pltpu.sync_copy(qidx_hbm.at[wid], qidx_v)

    copies = []
    for j in range(NCH):
        dst = pl.ds(j * CH, CH)
        copies.append(pltpu.async_copy(theta_hbm.at[sidx_v.at[j]],
                                       theta_v.at[dst], sem))
        copies.append(pltpu.async_copy(alpha_hbm.at[qidx_v.at[j]],
                                       alpha_v.at[dst], sem))
        copies.append(pltpu.async_copy(beta_hbm.at[qidx_v.at[j]],
                                       beta_v.at[dst], sem))
    for cp in copies:
        cp.wait()

    def block(blk, carry):
        base_row = blk * D
        bvec = beta_v[pl.ds(base_row, D)]
        for j in range(D):
            i = base_row + j
            t = theta_v[i, :]
            a = alpha_v[i, :]
            x = 1.702 * (a * (t - bvec[j]))
            out_v[i, :] = 1.0 / (1.0 + jnp.exp(-x))
        return carry

    lax.fori_loop(0, ROWS // D, block, 0)

    pltpu.sync_copy(out_v, out_hbm.at[pl.ds(base, ROWS)])


@functools.partial(jax.jit, static_argnums=())
def kernel(theta_table, alpha_table, beta_table, student_ids, question_ids):
    sidx = student_ids.astype(jnp.int32).reshape(NW, NCH, CH)
    qidx = question_ids.astype(jnp.int32).reshape(NW, NCH, CH)
    beta_flat = beta_table.reshape(-1)

    mesh = plsc.VectorSubcoreMesh(core_axis_name="c", subcore_axis_name="s")
    run = pl.kernel(
        _irt_body,
        mesh=mesh,
        out_type=jax.ShapeDtypeStruct((B, D), jnp.float32),
        compiler_params=pltpu.CompilerParams(use_tc_tiling_on_sc=False),
        scratch_types=[
            pltpu.VMEM((NCH, CH), jnp.int32),
            pltpu.VMEM((NCH, CH), jnp.int32),
            pltpu.VMEM((ROWS, D), jnp.float32),
            pltpu.VMEM((ROWS, D), jnp.float32),
            pltpu.VMEM((ROWS,), jnp.float32),
            pltpu.VMEM((ROWS, D), jnp.float32),
            pltpu.SemaphoreType.DMA,
        ],
    )
    return run(theta_table, alpha_table, beta_flat, sidx, qidx)
